# Initial kernel scaffold; baseline (speedup 1.0000x reference)
import jax
import jax.numpy as jnp
from jax.experimental import pallas as pl


def kernel(design_2d, obs_ids, mixer_ids, normal_rvs, means, std_deviations):
    raise NotImplementedError("write your pallas kernel here")


# speedup vs baseline: 3.6887x; 3.6887x over previous
import functools

import jax
import jax.numpy as jnp
from jax import lax
from jax.experimental import pallas as pl
from jax.experimental.pallas import tpu as pltpu
from jax.experimental.pallas import tpu_sc as plsc

_NCOLS = 23
_NMIX = 20000
_NOBS = 26666
_ND = 16
_NVARS = 12
_MIX_COLS = (0, 2, 4, 8, 1, 3, 6, 9, 5, 7, 21, 22)
_NLOG = 8
_LSTD = 0.8326

_N = 160000
_W = 32
_C = 128
_CHUNKS_PW = 40
_ROWS_PW = _C * _CHUNKS_PW
_NPAD = _W * _ROWS_PW
_PAD_OBS = _NOBS
_SEGROWS = 209 * _C
_DUMP = _SEGROWS - 8
_NEG = -3.0e38

_mesh = plsc.VectorSubcoreMesh(core_axis_name="c", subcore_axis_name="s",
                               num_cores=2, num_subcores=16)
_sc_params = pltpu.CompilerParams(use_tc_tiling_on_sc=False,
                                  needs_layout_passes=False)


def _wid():
    return lax.axis_index("s") * 2 + lax.axis_index("c")


def _rowdat_body(d_ref, mvec_ref, out_ref):
    d = d_ref[...]
    mvec = mvec_ref[...]
    base = jnp.sum(d * mvec, axis=1, keepdims=True)
    cols = [d[:, c:c + 1] for c in _MIX_COLS]
    pad = jnp.zeros((d.shape[0], 3), jnp.float32)
    out_ref[...] = jnp.concatenate(cols + [base, pad], axis=1)


def _gtab_body(rvs_ref, params_ref, out_ref):
    i = pl.program_id(0)
    mu = params_ref[i, 0]
    std = params_ref[i, 1]
    islog = params_ref[i, 2]
    gen = mu + std * rvs_ref[...]
    gexp = jnp.exp(jnp.clip(gen, -700.0, 700.0))
    out_ref[...] = jnp.where(islog > 0.5, gexp, gen)


def _mean_body(p_ref, out_ref):
    out_ref[...] = jnp.sum(p_ref[...], axis=1, keepdims=True) * (1.0 / _ND)


@functools.partial(
    pl.kernel,
    out_type=(jax.ShapeDtypeStruct((_NPAD, _ND), jnp.float32),
              jax.ShapeDtypeStruct((_SEGROWS, _ND), jnp.float32),
              jax.ShapeDtypeStruct((64, _ND), jnp.float32)),
    mesh=_mesh,
    compiler_params=_sc_params,
    scratch_types=[
        pltpu.VMEM((_C, _ND), jnp.float32),
        pltpu.VMEM((_C,), jnp.int32),
        pltpu.VMEM((_C,), jnp.int32),
        pltpu.VMEM((_C,), jnp.int32),
        pltpu.VMEM((_C,), jnp.int32),
        pltpu.VMEM((_NVARS, _C), jnp.int32),
        pltpu.VMEM((_NVARS, _C, _ND), jnp.float32),
        pltpu.VMEM((_C, _ND), jnp.float32),
        pltpu.VMEM((_C, _ND), jnp.float32),
        pltpu.VMEM((_C,), jnp.int32),
        pltpu.VMEM((_ND,), jnp.int32),
        pltpu.VMEM((_ND,), jnp.int32),
        pltpu.VMEM((2, _ND), jnp.float32),
        pltpu.SemaphoreType.DMA,
    ],
)
def _sc_util(rowdat_hbm, mid_hbm, obs_hbm, obsn_hbm, obsp_hbm, gtab_hbm,
             ff_hbm, lf_hbm, u_hbm, seg1_hbm, bparts_hbm,
             rowbuf, midbuf, obsbuf, nbuf, pbuf2, idxbuf, gbuf, ubuf,
             emitbuf, eidxbuf, ffbuf, lfbuf, bpbuf, sem):
    w = _wid()
    pltpu.sync_copy(ff_hbm.at[w], ffbuf)
    pltpu.sync_copy(lf_hbm.at[w], lfbuf)
    ffull = ffbuf[...]
    lfull = lfbuf[...]

    def chunk_body(k, carry):
        base = (w * _CHUNKS_PW + k) * _C
        pltpu.sync_copy(rowdat_hbm.at[pl.ds(base, _C)], rowbuf)
        pltpu.sync_copy(mid_hbm.at[pl.ds(base, _C)], midbuf)
        pltpu.sync_copy(obs_hbm.at[pl.ds(base, _C)], obsbuf)
        pltpu.sync_copy(obsn_hbm.at[pl.ds(base, _C)], nbuf)
        pltpu.sync_copy(obsp_hbm.at[pl.ds(base, _C)], pbuf2)
        for i in range(_NVARS):
            for s in range(_C // _ND):
                idxbuf[i, pl.ds(s * _ND, _ND)] = (
                    midbuf[pl.ds(s * _ND, _ND)] + i * _NMIX)
        copies = [pltpu.async_copy(gtab_hbm.at[idxbuf.at[i]], gbuf.at[i], sem)
                  for i in range(_NVARS)]
        for cp in copies:
            cp.wait()

        def group_body(s, gc):
            cur, fmax = gc
            ov = obsbuf[pl.ds(s * _ND, _ND)]
            nv = nbuf[pl.ds(s * _ND, _ND)]
            pv = pbuf2[pl.ds(s * _ND, _ND)]
            er = (ov != nv) & (ov != ffull) & (ov != lfull)
            eidxbuf[pl.ds(s * _ND, _ND)] = jnp.where(
                er, ov, jnp.full((_ND,), _DUMP, jnp.int32))
            sgate = (ov == pv).astype(jnp.float32)
            fgate = (ov == ffull).astype(jnp.float32)
            for j in range(_ND):
                r = s * _ND + j
                rv = rowbuf[r, :]
                acc = rv[_NVARS] + jnp.zeros((_ND,), jnp.float32)
                for i in range(_NVARS):
                    acc = acc + rv[i] * gbuf[i, r, :]
                acc = jnp.clip(acc, -700.0, 700.0)
                ubuf[r, :] = acc
                cur = jnp.maximum(acc, cur + (sgate[j] - 1.0) * 1.0e38)
                emitbuf[r, :] = cur
                fmax = jnp.maximum(fmax, acc + (fgate[j] - 1.0) * 1.0e38)
            return (cur, fmax)

        carry = lax.fori_loop(0, _C // _ND, group_body, carry)
        pltpu.async_copy(emitbuf, seg1_hbm.at[eidxbuf], sem).wait()
        pltpu.sync_copy(ubuf, u_hbm.at[pl.ds(base, _C)])
        return carry

    init = (jnp.full((_ND,), _NEG, jnp.float32),
            jnp.full((_ND,), _NEG, jnp.float32))
    last_max, first_max = lax.fori_loop(0, _CHUNKS_PW, chunk_body, init)
    bpbuf[0, :] = first_max
    bpbuf[1, :] = last_max
    pltpu.sync_copy(bpbuf, bparts_hbm.at[pl.ds(2 * w, 2)])


@functools.partial(
    pl.kernel,
    out_type=jax.ShapeDtypeStruct((_SEGROWS, _ND), jnp.float32),
    mesh=_mesh,
    compiler_params=_sc_params,
    scratch_types=[
        pltpu.VMEM((_C, _ND), jnp.float32),
        pltpu.VMEM((64, _ND), jnp.float32),
        pltpu.VMEM((64,), jnp.int32),
        pltpu.VMEM((64,), jnp.int32),
        pltpu.VMEM((64,), jnp.int32),
        pltpu.VMEM((64, _ND), jnp.float32),
        pltpu.VMEM((64,), jnp.int32),
        pltpu.SemaphoreType.DMA,
    ],
)
def _sc_fix(seg1_hbm, bparts_hbm, bids_hbm, bidn_hbm, bidp_hbm, seg2_hbm,
            cbuf, bpbuf, bidbuf, bnbuf, bpbuf2, em2, ei2, sem):
    w = _wid()

    @pl.when(w == 0)
    def _():
        def copy_chunk(k, c):
            pltpu.sync_copy(seg1_hbm.at[pl.ds(k * _C, _C)], cbuf)
            pltpu.sync_copy(cbuf, seg2_hbm.at[pl.ds(k * _C, _C)])
            return c

        lax.fori_loop(0, _SEGROWS // _C, copy_chunk, 0)
        pltpu.sync_copy(bparts_hbm, bpbuf)
        pltpu.sync_copy(bids_hbm, bidbuf)
        pltpu.sync_copy(bidn_hbm, bnbuf)
        pltpu.sync_copy(bidp_hbm, bpbuf2)
        cur = jnp.full((_ND,), _NEG, jnp.float32)
        for g in range(4):
            bv = bidbuf[pl.ds(g * _ND, _ND)]
            nv = bnbuf[pl.ds(g * _ND, _ND)]
            pv = bpbuf2[pl.ds(g * _ND, _ND)]
            ei2[pl.ds(g * _ND, _ND)] = jnp.where(
                bv != nv, bv, jnp.full((_ND,), _DUMP, jnp.int32))
            sgate = (bv == pv).astype(jnp.float32)
            for j in range(_ND):
                val = bpbuf[g * _ND + j, :]
                cur = jnp.maximum(val, cur + (sgate[j] - 1.0) * 1.0e38)
                em2[g * _ND + j, :] = cur
        pltpu.async_copy(em2, seg2_hbm.at[ei2], sem).wait()


@functools.partial(
    pl.kernel,
    out_type=(jax.ShapeDtypeStruct((_SEGROWS, _ND), jnp.float32),
              jax.ShapeDtypeStruct((_SEGROWS, _ND), jnp.float32)),
    mesh=_mesh,
    compiler_params=_sc_params,
    scratch_types=[
        pltpu.VMEM((_C, _ND), jnp.float32),
        pltpu.VMEM((_C,), jnp.int32),
        pltpu.VMEM((_C, _ND), jnp.float32),
        pltpu.VMEM((_C, _ND), jnp.float32),
        pltpu.VMEM_SHARED((_SEGROWS, _ND), jnp.float32),
        pltpu.SemaphoreType.DMA,
    ],
)
def _sc_sum(u_hbm, obs_hbm, seg2_hbm, zeros_hbm, denA_hbm, denB_hbm,
            ubuf, obsbuf, mxbuf, ebuf, shared, sem):
    w = _wid()
    s_ax = lax.axis_index("s")
    c_ax = lax.axis_index("c")

    @pl.when(s_ax == 0)
    def _():
        pltpu.sync_copy(zeros_hbm, shared)

    plsc.subcore_barrier()

    def chunk_body(k, carry):
        base = (w * _CHUNKS_PW + k) * _C
        pltpu.sync_copy(u_hbm.at[pl.ds(base, _C)], ubuf)
        pltpu.sync_copy(obs_hbm.at[pl.ds(base, _C)], obsbuf)
        pltpu.async_copy(seg2_hbm.at[obsbuf], mxbuf, sem).wait()

        def row(r, c2):
            ebuf[r, :] = jnp.exp(ubuf[r, :] - mxbuf[r, :])
            return c2

        lax.fori_loop(0, _C, row, 0)
        pltpu.sync_copy(ebuf, shared.at[obsbuf], add=True)
        return carry

    lax.fori_loop(0, _CHUNKS_PW, chunk_body, 0)
    plsc.subcore_barrier()

    @pl.when((s_ax == 0) & (c_ax == 0))
    def _():
        pltpu.sync_copy(shared, denA_hbm)

    @pl.when((s_ax == 0) & (c_ax == 1))
    def _():
        pltpu.sync_copy(shared, denB_hbm)


@functools.partial(
    pl.kernel,
    out_type=jax.ShapeDtypeStruct((_NPAD, _ND), jnp.float32),
    mesh=_mesh,
    compiler_params=_sc_params,
    scratch_types=[
        pltpu.VMEM((_C, _ND), jnp.float32),
        pltpu.VMEM((_C,), jnp.int32),
        pltpu.VMEM((_C, _ND), jnp.float32),
        pltpu.VMEM((_C, _ND), jnp.float32),
        pltpu.VMEM((_C, _ND), jnp.float32),
        pltpu.VMEM((_C, _ND), jnp.float32),
        pltpu.SemaphoreType.DMA,
    ],
)
def _sc_probs(u_hbm, obs_hbm, seg2_hbm, denA_hbm, denB_hbm, probs_hbm,
              ubuf, obsbuf, mxbuf, dAbuf, dBbuf, pbuf, sem):
    w = _wid()

    def chunk_body(k, carry):
        base = (w * _CHUNKS_PW + k) * _C
        pltpu.sync_copy(u_hbm.at[pl.ds(base, _C)], ubuf)
        pltpu.sync_copy(obs_hbm.at[pl.ds(base, _C)], obsbuf)
        c1 = pltpu.async_copy(seg2_hbm.at[obsbuf], mxbuf, sem)
        c2 = pltpu.async_copy(denA_hbm.at[obsbuf], dAbuf, sem)
        c3 = pltpu.async_copy(denB_hbm.at[obsbuf], dBbuf, sem)
        c1.wait()
        c2.wait()
        c3.wait()

        def row(r, c4):
            e = jnp.exp(ubuf[r, :] - mxbuf[r, :])
            p = e / (dAbuf[r, :] + dBbuf[r, :])
            pbuf[r, :] = jnp.clip(p, 1e-30, 1.0 - 1e-7)
            return c4

        lax.fori_loop(0, _C, row, 0)
        pltpu.sync_copy(pbuf, probs_hbm.at[pl.ds(base, _C)])
        return carry

    lax.fori_loop(0, _CHUNKS_PW, chunk_body, 0)


def kernel(design_2d, obs_ids, mixer_ids, normal_rvs, means, std_deviations):
    f32 = jnp.float32
    i32 = jnp.int32
    mix_cols = jnp.asarray(_MIX_COLS, dtype=i32)
    stds = jnp.concatenate([jnp.full((_NLOG,), _LSTD, f32), std_deviations])
    mu = means[mix_cols]
    islog = (jnp.arange(_NVARS) < _NLOG).astype(f32)
    params = jnp.stack([mu, stds, islog], axis=1)
    mvec = means.at[mix_cols].set(0.0)[None, :]

    npad = _NPAD - _N
    design_p = jnp.pad(design_2d, ((0, npad), (0, 0)))
    obs_p = jnp.pad(obs_ids.astype(i32), (0, npad), constant_values=_PAD_OBS)
    mid_p = jnp.pad(mixer_ids.astype(i32), (0, npad))
    obs_n = jnp.concatenate([obs_p[1:], jnp.full((1,), -2, i32)])
    obs_v = jnp.concatenate([jnp.full((1,), -3, i32), obs_p[:-1]])

    firsts = obs_p[:: _ROWS_PW]
    lasts = obs_p[_ROWS_PW - 1:: _ROWS_PW]
    fl = jnp.stack([firsts, lasts], axis=1).reshape(64)
    fln = jnp.concatenate([fl[1:], jnp.full((1,), -2, i32)])
    flp = jnp.concatenate([jnp.full((1,), -3, i32), fl[:-1]])
    ff = jnp.broadcast_to(firsts[:, None], (_W, _ND))
    lf = jnp.broadcast_to(lasts[:, None], (_W, _ND))
    zerotab = jnp.zeros((_SEGROWS, _ND), f32)

    blk = 4096
    rowdat = pl.pallas_call(
        _rowdat_body,
        grid=(_NPAD // blk,),
        in_specs=[
            pl.BlockSpec((blk, _NCOLS), lambda i: (i, 0)),
            pl.BlockSpec((1, _NCOLS), lambda i: (0, 0)),
        ],
        out_specs=pl.BlockSpec((blk, _ND), lambda i: (i, 0)),
        out_shape=jax.ShapeDtypeStruct((_NPAD, _ND), f32),
    )(design_p, mvec)

    gtab = pl.pallas_call(
        _gtab_body,
        grid=(_NVARS,),
        in_specs=[
            pl.BlockSpec((1, _NMIX, _ND), lambda i: (i, 0, 0)),
            pl.BlockSpec(memory_space=pltpu.SMEM),
        ],
        out_specs=pl.BlockSpec((1, _NMIX, _ND), lambda i: (i, 0, 0)),
        out_shape=jax.ShapeDtypeStruct((_NVARS, _NMIX, _ND), f32),
    )(normal_rvs, params)
    gtab = gtab.reshape(_NVARS * _NMIX, _ND)

    u, seg1, bparts = _sc_util(rowdat, mid_p, obs_p, obs_n, obs_v, gtab,
                               ff, lf)
    seg2 = _sc_fix(seg1, bparts, fl, fln, flp)
    denA, denB = _sc_sum(u, obs_p, seg2, zerotab)
    probs16 = _sc_probs(u, obs_p, seg2, denA, denB)

    mblk = 4000
    out = pl.pallas_call(
        _mean_body,
        grid=(_N // mblk,),
        in_specs=[pl.BlockSpec((mblk, _ND), lambda i: (i, 0))],
        out_specs=pl.BlockSpec((mblk, 1), lambda i: (i, 0)),
        out_shape=jax.ShapeDtypeStruct((_N, 1), f32),
    )(probs16[:_N])
    return out[:, 0]
